# Initial kernel scaffold; baseline (speedup 1.0000x reference)
import jax
import jax.numpy as jnp
from jax.experimental import pallas as pl


def kernel(q, k, positions):
    raise NotImplementedError("write your pallas kernel here")


# speedup vs baseline: 2.7011x; 2.7011x over previous
import functools

import jax
import jax.numpy as jnp
import numpy as np
from jax.experimental import pallas as pl
from jax.experimental.pallas import tpu as pltpu

NT = 4096
NH = 64
D = 128
BT = 64


def _hadamard(d):
    h = np.array([[1.0]], dtype=np.float32)
    base = np.array([[1.0, 1.0], [1.0, -1.0]], dtype=np.float32)
    while h.shape[0] < d:
        h = np.kron(h, base)
    return h


_H_BF16 = jnp.asarray(_hadamard(D), dtype=jnp.bfloat16)
_SCALE = float(D) ** -0.5


def _rotate_body(q_ref, k_ref, h_ref, qo_ref, ko_ref):
    h = h_ref[...]
    q = q_ref[...].reshape(BT * NH, D)
    acc = jax.lax.dot_general(
        q, h, (((1,), (0,)), ((), ())), preferred_element_type=jnp.float32
    )
    qo_ref[...] = (acc * _SCALE).reshape(BT, NH, D)
    kacc = jax.lax.dot_general(
        k_ref[...], h, (((1,), (0,)), ((), ())), preferred_element_type=jnp.float32
    )
    ko_ref[...] = kacc * _SCALE


@jax.jit
def _rotate(q, k):
    grid = (NT // BT,)
    return pl.pallas_call(
        _rotate_body,
        grid=grid,
        in_specs=[
            pl.BlockSpec((BT, NH, D), lambda i: (i, 0, 0)),
            pl.BlockSpec((BT, D), lambda i: (i, 0)),
            pl.BlockSpec((D, D), lambda i: (0, 0)),
        ],
        out_specs=[
            pl.BlockSpec((BT, NH, D), lambda i: (i, 0, 0)),
            pl.BlockSpec((BT, D), lambda i: (i, 0)),
        ],
        out_shape=[
            jax.ShapeDtypeStruct((NT, NH, D), jnp.float32),
            jax.ShapeDtypeStruct((NT, D), jnp.float32),
        ],
        compiler_params=pltpu.CompilerParams(
            dimension_semantics=("arbitrary",),
        ),
    )(q, k, _H_BF16)


def kernel(q, k, positions):
    del positions
    qo, ko = _rotate(q, k)
    return (qo, ko)


# speedup vs baseline: 3.4012x; 1.2592x over previous
import functools

import jax
import jax.numpy as jnp
import numpy as np
from jax.experimental import pallas as pl
from jax.experimental.pallas import tpu as pltpu

NT = 4096
NH = 64
D = 128
BT = 128


def _hadamard(d):
    h = np.array([[1.0]], dtype=np.float32)
    base = np.array([[1.0, 1.0], [1.0, -1.0]], dtype=np.float32)
    while h.shape[0] < d:
        h = np.kron(h, base)
    return h


_H_BF16 = jnp.asarray(_hadamard(D), dtype=jnp.bfloat16)
_SCALE = float(D) ** -0.5


def _rotate_body(q_ref, k_ref, h_ref, qo_ref, ko_ref):
    h = h_ref[...]
    q = q_ref[...].reshape(BT * NH, D)
    acc = jax.lax.dot_general(
        q, h, (((1,), (0,)), ((), ())), preferred_element_type=jnp.float32
    )
    qo_ref[...] = (acc * _SCALE).reshape(BT, NH, D)
    kacc = jax.lax.dot_general(
        k_ref[...], h, (((1,), (0,)), ((), ())), preferred_element_type=jnp.float32
    )
    ko_ref[...] = kacc * _SCALE


@jax.jit
def _rotate(q, k):
    grid = (NT // BT,)
    return pl.pallas_call(
        _rotate_body,
        grid=grid,
        in_specs=[
            pl.BlockSpec((BT, NH, D), lambda i: (i, 0, 0)),
            pl.BlockSpec((BT, D), lambda i: (i, 0)),
            pl.BlockSpec((D, D), lambda i: (0, 0)),
        ],
        out_specs=[
            pl.BlockSpec((BT, NH, D), lambda i: (i, 0, 0)),
            pl.BlockSpec((BT, D), lambda i: (i, 0)),
        ],
        out_shape=[
            jax.ShapeDtypeStruct((NT, NH, D), jnp.float32),
            jax.ShapeDtypeStruct((NT, D), jnp.float32),
        ],
        compiler_params=pltpu.CompilerParams(
            dimension_semantics=("arbitrary",),
        ),
    )(q, k, _H_BF16)


def kernel(q, k, positions):
    del positions
    qo, ko = _rotate(q, k)
    return (qo, ko)


# speedup vs baseline: 3.6094x; 1.0612x over previous
import functools

import jax
import jax.numpy as jnp
import numpy as np
from jax.experimental import pallas as pl
from jax.experimental.pallas import tpu as pltpu

NT = 4096
NH = 64
D = 128
BT = 256


def _hadamard(d):
    h = np.array([[1.0]], dtype=np.float32)
    base = np.array([[1.0, 1.0], [1.0, -1.0]], dtype=np.float32)
    while h.shape[0] < d:
        h = np.kron(h, base)
    return h


_H_BF16 = jnp.asarray(_hadamard(D), dtype=jnp.bfloat16)
_SCALE = float(D) ** -0.5


def _rotate_body(q_ref, k_ref, h_ref, qo_ref, ko_ref):
    h = h_ref[...]
    q = q_ref[...].reshape(BT * NH, D)
    acc = jax.lax.dot_general(
        q, h, (((1,), (0,)), ((), ())), preferred_element_type=jnp.float32
    )
    qo_ref[...] = (acc * _SCALE).reshape(BT, NH, D)
    kacc = jax.lax.dot_general(
        k_ref[...], h, (((1,), (0,)), ((), ())), preferred_element_type=jnp.float32
    )
    ko_ref[...] = kacc * _SCALE


@jax.jit
def _rotate(q, k):
    grid = (NT // BT,)
    return pl.pallas_call(
        _rotate_body,
        grid=grid,
        in_specs=[
            pl.BlockSpec((BT, NH, D), lambda i: (i, 0, 0)),
            pl.BlockSpec((BT, D), lambda i: (i, 0)),
            pl.BlockSpec((D, D), lambda i: (0, 0)),
        ],
        out_specs=[
            pl.BlockSpec((BT, NH, D), lambda i: (i, 0, 0)),
            pl.BlockSpec((BT, D), lambda i: (i, 0)),
        ],
        out_shape=[
            jax.ShapeDtypeStruct((NT, NH, D), jnp.float32),
            jax.ShapeDtypeStruct((NT, D), jnp.float32),
        ],
        compiler_params=pltpu.CompilerParams(
            dimension_semantics=("arbitrary",),
        ),
    )(q, k, _H_BF16)


def kernel(q, k, positions):
    del positions
    qo, ko = _rotate(q, k)
    return (qo, ko)


# speedup vs baseline: 3.7156x; 1.0294x over previous
import functools

import jax
import jax.numpy as jnp
import numpy as np
from jax.experimental import pallas as pl
from jax.experimental.pallas import tpu as pltpu

NT = 4096
NH = 64
D = 128
BT = 512


def _hadamard(d):
    h = np.array([[1.0]], dtype=np.float32)
    base = np.array([[1.0, 1.0], [1.0, -1.0]], dtype=np.float32)
    while h.shape[0] < d:
        h = np.kron(h, base)
    return h


_H_BF16 = jnp.asarray(_hadamard(D), dtype=jnp.bfloat16)
_SCALE = float(D) ** -0.5


def _rotate_body(q_ref, k_ref, h_ref, qo_ref, ko_ref):
    h = h_ref[...]
    q = q_ref[...].reshape(BT * NH, D)
    acc = jax.lax.dot_general(
        q, h, (((1,), (0,)), ((), ())), preferred_element_type=jnp.float32
    )
    qo_ref[...] = (acc * _SCALE).reshape(BT, NH, D)
    kacc = jax.lax.dot_general(
        k_ref[...], h, (((1,), (0,)), ((), ())), preferred_element_type=jnp.float32
    )
    ko_ref[...] = kacc * _SCALE


@jax.jit
def _rotate(q, k):
    grid = (NT // BT,)
    return pl.pallas_call(
        _rotate_body,
        grid=grid,
        in_specs=[
            pl.BlockSpec((BT, NH, D), lambda i: (i, 0, 0)),
            pl.BlockSpec((BT, D), lambda i: (i, 0)),
            pl.BlockSpec((D, D), lambda i: (0, 0)),
        ],
        out_specs=[
            pl.BlockSpec((BT, NH, D), lambda i: (i, 0, 0)),
            pl.BlockSpec((BT, D), lambda i: (i, 0)),
        ],
        out_shape=[
            jax.ShapeDtypeStruct((NT, NH, D), jnp.float32),
            jax.ShapeDtypeStruct((NT, D), jnp.float32),
        ],
        compiler_params=pltpu.CompilerParams(
            dimension_semantics=("arbitrary",),
        ),
    )(q, k, _H_BF16)


def kernel(q, k, positions):
    del positions
    qo, ko = _rotate(q, k)
    return (qo, ko)


# speedup vs baseline: 3.7251x; 1.0026x over previous
import functools

import jax
import jax.numpy as jnp
import numpy as np
from jax.experimental import pallas as pl
from jax.experimental.pallas import tpu as pltpu

NT = 4096
NH = 64
D = 128
BT = 512


def _hadamard(d):
    h = np.array([[1.0]], dtype=np.float32)
    base = np.array([[1.0, 1.0], [1.0, -1.0]], dtype=np.float32)
    while h.shape[0] < d:
        h = np.kron(h, base)
    return h


_H_NP = _hadamard(D)
_SCALE = float(D) ** -0.5


def _rotate_body(q_ref, k_ref, h_ref, qo_ref, ko_ref):
    h = h_ref[...]
    q = q_ref[...].reshape(BT * NH, D)
    acc = jax.lax.dot_general(
        q, h, (((1,), (0,)), ((), ())), preferred_element_type=jnp.float32
    )
    qo_ref[...] = acc.reshape(BT, NH, D)
    ko_ref[...] = jax.lax.dot_general(
        k_ref[...], h, (((1,), (0,)), ((), ())), preferred_element_type=jnp.float32
    )


@jax.jit
def _rotate(q, k):
    grid = (NT // BT,)
    return pl.pallas_call(
        _rotate_body,
        grid=grid,
        in_specs=[
            pl.BlockSpec((BT, NH, D), lambda i: (i, 0, 0)),
            pl.BlockSpec((BT, D), lambda i: (i, 0)),
            pl.BlockSpec((D, D), lambda i: (0, 0)),
        ],
        out_specs=[
            pl.BlockSpec((BT, NH, D), lambda i: (i, 0, 0)),
            pl.BlockSpec((BT, D), lambda i: (i, 0)),
        ],
        out_shape=[
            jax.ShapeDtypeStruct((NT, NH, D), jnp.float32),
            jax.ShapeDtypeStruct((NT, D), jnp.float32),
        ],
        compiler_params=pltpu.CompilerParams(
            dimension_semantics=("arbitrary",),
        ),
    )(q, k, jnp.asarray(_H_NP * _SCALE, dtype=jnp.bfloat16))


def kernel(q, k, positions):
    del positions
    qo, ko = _rotate(q, k)
    return (qo, ko)
